# speedup vs baseline: 1.1573x; 1.1573x over previous
import functools

import jax
import jax.numpy as jnp
from jax import lax
from jax.experimental import pallas as pl
from jax.experimental.pallas import tpu as pltpu
from jax.experimental.pallas import tpu_sc as plsc

VOCAB = 50257
N_EMBD = 768
BLOCK = 2048
B, T = 4, 2048

NC, NS, L = 2, 16, 16
NW = NC * NS
TPW = T // NW
VECS = N_EMBD // L

_mesh = plsc.VectorSubcoreMesh(core_axis_name="c", subcore_axis_name="s")


@functools.partial(
    pl.kernel,
    mesh=_mesh,
    out_type=jax.ShapeDtypeStruct((B * T, N_EMBD), jnp.float32),
    scratch_types=[
        pltpu.VMEM((TPW,), jnp.int32),
        pltpu.VMEM((TPW, N_EMBD), jnp.float32),
        pltpu.VMEM((TPW, N_EMBD), jnp.float32),
        pltpu.SemaphoreType.DMA,
    ],
)
def _embed(idx_hbm, wte_hbm, wpe_hbm, out_hbm, idx_v, wpe_v, tok_v, sem):
    wid = lax.axis_index("s") * NC + lax.axis_index("c")
    t0 = wid * TPW
    pltpu.sync_copy(wpe_hbm.at[pl.ds(t0, TPW)], wpe_v)

    def row_add(i, carry):
        for j in range(VECS):
            sl = pl.ds(j * L, L)
            tok_v[i, sl] = tok_v[i, sl] + wpe_v[i, sl]
        return carry

    for b in range(B):
        base = b * T + t0
        pltpu.sync_copy(idx_hbm.at[pl.ds(base, TPW)], idx_v)
        pltpu.async_copy(wte_hbm.at[idx_v], tok_v, sem).wait()
        lax.fori_loop(0, TPW, row_add, 0)
        pltpu.sync_copy(tok_v, out_hbm.at[pl.ds(base, TPW)])


def kernel(idx, wte, wpe):
    flat = _embed(idx.reshape(-1).astype(jnp.int32), wte, wpe)
    return flat.reshape(B, T, N_EMBD)
